# baseline (speedup 1.0000x reference)
import functools

import jax
import jax.numpy as jnp
from jax import lax
from jax.experimental import pallas as pl
from jax.experimental.pallas import tpu as pltpu
from jax.experimental.pallas import tpu_sc as plsc

N = 10000
D = 128
E = 320000
NPAD = 10240
NC, NS = 2, 16
NW = NC * NS
CHUNK = 120
NCHUNK = 84
EPW = CHUNK * NCHUNK
EPAD = NW * EPW - E
ROWS_PER_TILE = NPAD // NS

_mesh = plsc.VectorSubcoreMesh(core_axis_name="c", subcore_axis_name="s")


@functools.partial(
    pl.kernel,
    out_type=jax.ShapeDtypeStruct((NC, NPAD), jnp.float32),
    mesh=_mesh,
    scratch_types=[
        pltpu.VMEM((2, CHUNK), jnp.int32),
        pltpu.VMEM((2, CHUNK), jnp.int32),
        pltpu.VMEM((2, CHUNK), jnp.int32),
        pltpu.VMEM((128,), jnp.float32),
        pltpu.VMEM_SHARED((NPAD,), jnp.float32),
        pltpu.SemaphoreType.DMA,
        pltpu.SemaphoreType.DMA,
        pltpu.SemaphoreType.DMA,
        pltpu.SemaphoreType.DMA,
    ],
)
def _deg_kernel(ei_hbm, z1_hbm, out_hbm, idx0, idx1, idx2, onesv, acc,
                si0, si1, si2, semz):
    c = lax.axis_index("c")
    s = lax.axis_index("s")
    ch0 = (c * NS + s) * NCHUNK
    pltpu.async_copy(z1_hbm, acc.at[pl.ds(s * ROWS_PER_TILE, ROWS_PER_TILE)],
                     semz)
    pltpu.async_copy(ei_hbm.at[ch0], idx0, si0)
    pltpu.async_copy(ei_hbm.at[ch0 + 1], idx1, si1)
    pltpu.async_copy(ei_hbm.at[ch0 + 2], idx2, si2)
    for j in range(128 // 16):
        onesv[pl.ds(16 * j, 16)] = jnp.ones((16,), jnp.float32)
    pltpu.make_async_copy(
        z1_hbm, acc.at[pl.ds(s * ROWS_PER_TILE, ROWS_PER_TILE)], semz).wait()
    plsc.subcore_barrier()

    def halfstep(i, idxb, sib):
        pltpu.make_async_copy(ei_hbm.at[ch0 + i], idxb, sib).wait()
        pltpu.sync_copy(onesv.at[pl.ds(0, CHUNK)], acc.at[idxb.at[1]],
                        add=True)

        @pl.when(i + 3 < NCHUNK)
        def _():
            pltpu.async_copy(ei_hbm.at[ch0 + i + 3], idxb, sib)

    def step(j, _):
        halfstep(3 * j, idx0, si0)
        halfstep(3 * j + 1, idx1, si1)
        halfstep(3 * j + 2, idx2, si2)
        return ()

    lax.fori_loop(0, NCHUNK // 3, step, ())
    plsc.subcore_barrier()
    r0 = s * ROWS_PER_TILE
    pltpu.sync_copy(acc.at[pl.ds(r0, ROWS_PER_TILE)],
                    out_hbm.at[c, pl.ds(r0, ROWS_PER_TILE)])


@functools.partial(
    pl.kernel,
    out_type=jax.ShapeDtypeStruct((NC, NPAD, D), jnp.float32),
    mesh=_mesh,
    scratch_types=[
        pltpu.VMEM((2, CHUNK), jnp.int32),
        pltpu.VMEM((2, CHUNK), jnp.int32),
        pltpu.VMEM((2, CHUNK), jnp.int32),
        pltpu.VMEM((CHUNK, D), jnp.float32),
        pltpu.VMEM((CHUNK, D), jnp.float32),
        pltpu.VMEM((CHUNK, D), jnp.float32),
        pltpu.VMEM_SHARED((NPAD, D), jnp.float32),
        pltpu.SemaphoreType.DMA,
        pltpu.SemaphoreType.DMA,
        pltpu.SemaphoreType.DMA,
        pltpu.SemaphoreType.DMA,
        pltpu.SemaphoreType.DMA,
        pltpu.SemaphoreType.DMA,
        pltpu.SemaphoreType.DMA,
    ],
)
def _scatter_kernel(g_hbm, ei_hbm, z2_hbm, out_hbm,
                    idx0, idx1, idx2, rows0, rows1, rows2, acc,
                    sg0, sg1, sg2, si0, si1, si2, semz):
    c = lax.axis_index("c")
    s = lax.axis_index("s")
    ch0 = (c * NS + s) * NCHUNK
    for j in range(ROWS_PER_TILE // 128):
        pltpu.async_copy(
            z2_hbm, acc.at[pl.ds(s * ROWS_PER_TILE + j * 128, 128), :], semz)
    pltpu.async_copy(ei_hbm.at[ch0], idx0, si0)
    pltpu.async_copy(ei_hbm.at[ch0 + 1], idx1, si1)
    pltpu.async_copy(ei_hbm.at[ch0 + 2], idx2, si2)
    pltpu.make_async_copy(ei_hbm.at[ch0], idx0, si0).wait()
    pltpu.async_copy(g_hbm.at[idx0.at[0]], rows0, sg0)
    pltpu.make_async_copy(ei_hbm.at[ch0 + 1], idx1, si1).wait()
    pltpu.async_copy(g_hbm.at[idx1.at[0]], rows1, sg1)
    for j in range(ROWS_PER_TILE // 128):
        pltpu.make_async_copy(
            z2_hbm, acc.at[pl.ds(s * ROWS_PER_TILE + j * 128, 128), :],
            semz).wait()
    plsc.subcore_barrier()

    def halfstep(i, idxb, rowsb, sgb, sib, idx2b, rows2b, sg2b, si2b):
        pltpu.make_async_copy(g_hbm.at[idxb.at[0]], rowsb, sgb).wait()

        @pl.when(i + 2 < NCHUNK)
        def _():
            pltpu.make_async_copy(ei_hbm.at[ch0 + i + 2], idx2b, si2b).wait()
            pltpu.async_copy(g_hbm.at[idx2b.at[0]], rows2b, sg2b)

        pltpu.sync_copy(rowsb, acc.at[idxb.at[1]], add=True)

        @pl.when(i + 3 < NCHUNK)
        def _():
            pltpu.async_copy(ei_hbm.at[ch0 + i + 3], idxb, sib)

    def step(j, _):
        i0 = 3 * j
        halfstep(i0, idx0, rows0, sg0, si0, idx2, rows2, sg2, si2)
        halfstep(i0 + 1, idx1, rows1, sg1, si1, idx0, rows0, sg0, si0)
        halfstep(i0 + 2, idx2, rows2, sg2, si2, idx1, rows1, sg1, si1)
        return ()

    lax.fori_loop(0, NCHUNK // 3, step, ())
    plsc.subcore_barrier()
    r0 = s * ROWS_PER_TILE
    pltpu.sync_copy(acc.at[pl.ds(r0, ROWS_PER_TILE), :],
                    out_hbm.at[c, pl.ds(r0, ROWS_PER_TILE), :])


_RB = 10240
_GRID = (N + _RB - 1) // _RB


def _dense1_body(x_ref, w_ref, d0_ref, d1_ref, g_ref):
    dis = lax.rsqrt(d0_ref[...] + d1_ref[...] + 1.0)
    h = jnp.dot(x_ref[...], w_ref[...], preferred_element_type=jnp.float32)
    g_ref[...] = h * dis[:, None]


def _dense2_body(a_ref, g_ref, d0_ref, d1_ref, b_ref, w_ref, out_ref):
    dis = lax.rsqrt(d0_ref[...] + d1_ref[...] + 1.0)
    z = dis[:, None] * (a_ref[0] + a_ref[1] + g_ref[...]) + b_ref[...][None, :]
    z = jnp.maximum(z, 0.0)
    out_ref[...] = jnp.dot(z, w_ref[...],
                           preferred_element_type=jnp.float32) * dis[:, None]


def _dense3_body(a_ref, g_ref, d0_ref, d1_ref, b_ref, out_ref):
    dis = lax.rsqrt(d0_ref[...] + d1_ref[...] + 1.0)
    out_ref[...] = (dis[:, None] * (a_ref[0] + a_ref[1] + g_ref[...])
                    + b_ref[...][None, :])


_row_spec = pl.BlockSpec((_RB, D), lambda i: (i, 0))
_deg_spec = pl.BlockSpec((_RB,), lambda i: (i,))
_a_spec = pl.BlockSpec((NC, _RB, D), lambda i: (0, i, 0))
_w_spec = pl.BlockSpec((D, D), lambda i: (0, 0))
_b_spec = pl.BlockSpec((D,), lambda i: (0,))

_dense1 = pl.pallas_call(
    _dense1_body,
    grid=_GRID,
    in_specs=[_row_spec, _w_spec, _deg_spec, _deg_spec],
    out_specs=_row_spec,
    out_shape=jax.ShapeDtypeStruct((N, D), jnp.float32),
)

_dense2 = pl.pallas_call(
    _dense2_body,
    grid=_GRID,
    in_specs=[_a_spec, _row_spec, _deg_spec, _deg_spec, _b_spec, _w_spec],
    out_specs=_row_spec,
    out_shape=jax.ShapeDtypeStruct((N, D), jnp.float32),
)

_dense3 = pl.pallas_call(
    _dense3_body,
    grid=_GRID,
    in_specs=[_a_spec, _row_spec, _deg_spec, _deg_spec, _b_spec],
    out_specs=_row_spec,
    out_shape=jax.ShapeDtypeStruct((N, D), jnp.float32),
)


def kernel(x, edge_index, W1, b1, W2, b2):
    ppw = EPAD // NW
    srcw = edge_index[0].astype(jnp.int32).reshape(NW, E // NW)
    dstw = edge_index[1].astype(jnp.int32).reshape(NW, E // NW)
    pad_src = jnp.broadcast_to(jnp.arange(ppw, dtype=jnp.int32), (NW, ppw))
    pad_dst = jnp.broadcast_to(N + jnp.arange(ppw, dtype=jnp.int32),
                               (NW, ppw))
    src = jnp.concatenate([srcw, pad_src], axis=1).reshape(NW * NCHUNK, CHUNK)
    dst = jnp.concatenate([dstw, pad_dst], axis=1).reshape(NW * NCHUNK, CHUNK)
    ei = jnp.stack([src, dst], axis=1)
    z1 = jnp.zeros((ROWS_PER_TILE,), jnp.float32)
    z2 = jnp.zeros((128, D), jnp.float32)

    degs = _deg_kernel(ei, z1)
    d0, d1 = degs[0], degs[1]

    g1 = _dense1(x, W1, d0, d1)
    a1 = _scatter_kernel(g1, ei, z2)
    g2 = _dense2(a1, g1, d0, d1, b1, W2)
    a2 = _scatter_kernel(g2, ei, z2)
    out = _dense3(a2, g2, d0, d1, b2)
    return out


# speedup vs baseline: 1.0118x; 1.0118x over previous
import functools

import jax
import jax.numpy as jnp
from jax import lax
from jax.experimental import pallas as pl
from jax.experimental.pallas import tpu as pltpu
from jax.experimental.pallas import tpu_sc as plsc

N = 10000
D = 128
E = 320000
NPAD = 10240
NC, NS = 2, 16
NW = NC * NS
CHUNK = 120
NCHUNK = 84
EPW = CHUNK * NCHUNK
EPAD = NW * EPW - E
ROWS_PER_TILE = NPAD // NS

_mesh = plsc.VectorSubcoreMesh(core_axis_name="c", subcore_axis_name="s")


@functools.partial(
    pl.kernel,
    out_type=jax.ShapeDtypeStruct((NC, NPAD), jnp.float32),
    mesh=_mesh,
    scratch_types=[
        pltpu.VMEM((2, CHUNK), jnp.int32),
        pltpu.VMEM((2, CHUNK), jnp.int32),
        pltpu.VMEM((2, CHUNK), jnp.int32),
        pltpu.VMEM((128,), jnp.float32),
        pltpu.VMEM_SHARED((NPAD,), jnp.float32),
        pltpu.SemaphoreType.DMA,
        pltpu.SemaphoreType.DMA,
        pltpu.SemaphoreType.DMA,
        pltpu.SemaphoreType.DMA,
    ],
)
def _deg_kernel(ei_hbm, z1_hbm, out_hbm, idx0, idx1, idx2, onesv, acc,
                si0, si1, si2, semz):
    c = lax.axis_index("c")
    s = lax.axis_index("s")
    ch0 = (c * NS + s) * NCHUNK
    pltpu.async_copy(z1_hbm, acc.at[pl.ds(s * ROWS_PER_TILE, ROWS_PER_TILE)],
                     semz)
    pltpu.async_copy(ei_hbm.at[ch0], idx0, si0)
    pltpu.async_copy(ei_hbm.at[ch0 + 1], idx1, si1)
    pltpu.async_copy(ei_hbm.at[ch0 + 2], idx2, si2)
    for j in range(128 // 16):
        onesv[pl.ds(16 * j, 16)] = jnp.ones((16,), jnp.float32)
    pltpu.make_async_copy(
        z1_hbm, acc.at[pl.ds(s * ROWS_PER_TILE, ROWS_PER_TILE)], semz).wait()
    plsc.subcore_barrier()

    def halfstep(i, idxb, sib):
        pltpu.make_async_copy(ei_hbm.at[ch0 + i], idxb, sib).wait()
        pltpu.sync_copy(onesv.at[pl.ds(0, CHUNK)], acc.at[idxb.at[1]],
                        add=True)

        @pl.when(i + 3 < NCHUNK)
        def _():
            pltpu.async_copy(ei_hbm.at[ch0 + i + 3], idxb, sib)

    def step(j, _):
        halfstep(3 * j, idx0, si0)
        halfstep(3 * j + 1, idx1, si1)
        halfstep(3 * j + 2, idx2, si2)
        return ()

    lax.fori_loop(0, NCHUNK // 3, step, ())
    plsc.subcore_barrier()
    r0 = s * ROWS_PER_TILE
    pltpu.sync_copy(acc.at[pl.ds(r0, ROWS_PER_TILE)],
                    out_hbm.at[c, pl.ds(r0, ROWS_PER_TILE)])


@functools.partial(
    pl.kernel,
    out_type=jax.ShapeDtypeStruct((NC, NPAD, D), jnp.float32),
    mesh=_mesh,
    scratch_types=[
        pltpu.VMEM((2, CHUNK), jnp.int32),
        pltpu.VMEM((2, CHUNK), jnp.int32),
        pltpu.VMEM((2, CHUNK), jnp.int32),
        pltpu.VMEM((CHUNK, D), jnp.float32),
        pltpu.VMEM((CHUNK, D), jnp.float32),
        pltpu.VMEM((CHUNK, D), jnp.float32),
        pltpu.VMEM_SHARED((NPAD, D), jnp.float32),
        pltpu.SemaphoreType.DMA,
        pltpu.SemaphoreType.DMA,
        pltpu.SemaphoreType.DMA,
        pltpu.SemaphoreType.DMA,
        pltpu.SemaphoreType.DMA,
        pltpu.SemaphoreType.DMA,
        pltpu.SemaphoreType.DMA,
    ],
)
def _scatter_kernel(g_hbm, ei_hbm, z2_hbm, out_hbm,
                    idx0, idx1, idx2, rows0, rows1, rows2, acc,
                    sg0, sg1, sg2, si0, si1, si2, semz):
    c = lax.axis_index("c")
    s = lax.axis_index("s")
    ch0 = (c * NS + s) * NCHUNK
    for j in range(ROWS_PER_TILE // 128):
        pltpu.async_copy(
            z2_hbm, acc.at[pl.ds(s * ROWS_PER_TILE + j * 128, 128), :], semz)
    pltpu.async_copy(ei_hbm.at[ch0], idx0, si0)
    pltpu.async_copy(ei_hbm.at[ch0 + 1], idx1, si1)
    pltpu.async_copy(ei_hbm.at[ch0 + 2], idx2, si2)
    pltpu.make_async_copy(ei_hbm.at[ch0], idx0, si0).wait()
    pltpu.async_copy(g_hbm.at[idx0.at[0]], rows0, sg0)
    pltpu.make_async_copy(ei_hbm.at[ch0 + 1], idx1, si1).wait()
    pltpu.async_copy(g_hbm.at[idx1.at[0]], rows1, sg1)
    for j in range(ROWS_PER_TILE // 128):
        pltpu.make_async_copy(
            z2_hbm, acc.at[pl.ds(s * ROWS_PER_TILE + j * 128, 128), :],
            semz).wait()
    plsc.subcore_barrier()

    def halfstep(i, idxb, rowsb, sgb, sib, idx2b, rows2b, sg2b, si2b):
        pltpu.make_async_copy(g_hbm.at[idxb.at[0]], rowsb, sgb).wait()

        @pl.when(i + 2 < NCHUNK)
        def _():
            pltpu.make_async_copy(ei_hbm.at[ch0 + i + 2], idx2b, si2b).wait()
            pltpu.async_copy(g_hbm.at[idx2b.at[0]], rows2b, sg2b)

        pltpu.sync_copy(rowsb, acc.at[idxb.at[1]], add=True)

        @pl.when(i + 3 < NCHUNK)
        def _():
            pltpu.async_copy(ei_hbm.at[ch0 + i + 3], idxb, sib)

    def step(j, _):
        i0 = 3 * j
        halfstep(i0, idx0, rows0, sg0, si0, idx2, rows2, sg2, si2)
        halfstep(i0 + 1, idx1, rows1, sg1, si1, idx0, rows0, sg0, si0)
        halfstep(i0 + 2, idx2, rows2, sg2, si2, idx1, rows1, sg1, si1)
        return ()

    lax.fori_loop(0, NCHUNK // 3, step, ())
    plsc.subcore_barrier()
    r0 = s * ROWS_PER_TILE
    pltpu.sync_copy(acc.at[pl.ds(r0, ROWS_PER_TILE), :],
                    out_hbm.at[c, pl.ds(r0, ROWS_PER_TILE), :])


_RB = 5120
_GRID = (N + _RB - 1) // _RB


def _dense1_body(x_ref, w_ref, d0_ref, d1_ref, g_ref):
    dis = lax.rsqrt(d0_ref[...] + d1_ref[...] + 1.0)
    h = jnp.dot(x_ref[...], w_ref[...], preferred_element_type=jnp.float32)
    g_ref[...] = h * dis[:, None]


def _dense2_body(a_ref, g_ref, d0_ref, d1_ref, b_ref, w_ref, out_ref):
    dis = lax.rsqrt(d0_ref[...] + d1_ref[...] + 1.0)
    z = dis[:, None] * (a_ref[0] + a_ref[1] + g_ref[...]) + b_ref[...][None, :]
    z = jnp.maximum(z, 0.0)
    out_ref[...] = jnp.dot(z, w_ref[...],
                           preferred_element_type=jnp.float32) * dis[:, None]


def _dense3_body(a_ref, g_ref, d0_ref, d1_ref, b_ref, out_ref):
    dis = lax.rsqrt(d0_ref[...] + d1_ref[...] + 1.0)
    out_ref[...] = (dis[:, None] * (a_ref[0] + a_ref[1] + g_ref[...])
                    + b_ref[...][None, :])


_row_spec = pl.BlockSpec((_RB, D), lambda i: (i, 0))
_deg_spec = pl.BlockSpec((_RB,), lambda i: (i,))
_a_spec = pl.BlockSpec((NC, _RB, D), lambda i: (0, i, 0))
_w_spec = pl.BlockSpec((D, D), lambda i: (0, 0))
_b_spec = pl.BlockSpec((D,), lambda i: (0,))

_dense1 = pl.pallas_call(
    _dense1_body,
    grid=_GRID,
    in_specs=[_row_spec, _w_spec, _deg_spec, _deg_spec],
    out_specs=_row_spec,
    out_shape=jax.ShapeDtypeStruct((N, D), jnp.float32),
)

_dense2 = pl.pallas_call(
    _dense2_body,
    grid=_GRID,
    in_specs=[_a_spec, _row_spec, _deg_spec, _deg_spec, _b_spec, _w_spec],
    out_specs=_row_spec,
    out_shape=jax.ShapeDtypeStruct((N, D), jnp.float32),
)

_dense3 = pl.pallas_call(
    _dense3_body,
    grid=_GRID,
    in_specs=[_a_spec, _row_spec, _deg_spec, _deg_spec, _b_spec],
    out_specs=_row_spec,
    out_shape=jax.ShapeDtypeStruct((N, D), jnp.float32),
)


def kernel(x, edge_index, W1, b1, W2, b2):
    ppw = EPAD // NW
    srcw = edge_index[0].astype(jnp.int32).reshape(NW, E // NW)
    dstw = edge_index[1].astype(jnp.int32).reshape(NW, E // NW)
    pad_src = jnp.broadcast_to(jnp.arange(ppw, dtype=jnp.int32), (NW, ppw))
    pad_dst = jnp.broadcast_to(N + jnp.arange(ppw, dtype=jnp.int32),
                               (NW, ppw))
    src = jnp.concatenate([srcw, pad_src], axis=1).reshape(NW * NCHUNK, CHUNK)
    dst = jnp.concatenate([dstw, pad_dst], axis=1).reshape(NW * NCHUNK, CHUNK)
    ei = jnp.stack([src, dst], axis=1)
    z1 = jnp.zeros((ROWS_PER_TILE,), jnp.float32)
    z2 = jnp.zeros((128, D), jnp.float32)

    degs = _deg_kernel(ei, z1)
    d0, d1 = degs[0], degs[1]

    g1 = _dense1(x, W1, d0, d1)
    a1 = _scatter_kernel(g1, ei, z2)
    g2 = _dense2(a1, g1, d0, d1, b1, W2)
    a2 = _scatter_kernel(g2, ei, z2)
    out = _dense3(a2, g2, d0, d1, b2)
    return out
